# Initial kernel scaffold; baseline (speedup 1.0000x reference)
import functools

import jax
import jax.numpy as jnp
from jax import lax
from jax.experimental import pallas as pl
from jax.experimental.pallas import tpu as pltpu
from jax.experimental.pallas import tpu_sc as plsc

B, T, D = 16, 512, 256
F = 256
MEL = 2048

NW = 32
ROWS_W = (B * MEL) // NW
CH = 128
SENT = B * T


_sc_mesh = plsc.VectorSubcoreMesh(core_axis_name="c", subcore_axis_name="s")


@functools.partial(
    pl.kernel,
    mesh=_sc_mesh,
    out_type=jax.ShapeDtypeStruct((B * MEL, D), jnp.float32),
    scratch_types=[
        pltpu.VMEM((T,), jnp.int32),
        pltpu.VMEM((ROWS_W,), jnp.int32),
        pltpu.VMEM((CH, D), jnp.float32),
        pltpu.VMEM((CH, D), jnp.float32),
        pltpu.SemaphoreType.DMA,
        pltpu.SemaphoreType.DMA,
    ],
)
def _expand_sc(x_hbm, tgt_hbm, out_hbm, tgt_v, idx_v, buf0, buf1, sem0, sem1):
    w = lax.axis_index("s") * 2 + lax.axis_index("c")
    b = w // 2
    half = w % 2
    base = half * ROWS_W

    pltpu.sync_copy(tgt_hbm.at[b], tgt_v)

    sent = jnp.full((16,), SENT, jnp.int32)
    for j in range(ROWS_W // 16):
        idx_v[pl.ds(j * 16, 16)] = sent

    iota16 = lax.iota(jnp.int32, 16)
    row0 = b * T
    carry = jnp.int32(0)
    for i in range(T // 16):
        t_chunk = tgt_v[pl.ds(i * 16, 16)]
        csum = plsc.cumsum(t_chunk)
        start = csum - t_chunk + carry
        val = jnp.full((16,), row0 + i * 16, jnp.int32) + iota16
        pos = start - base
        for k in range(3):
            pk = pos + k
            m = (t_chunk > k) & (pk >= 0) & (pk < ROWS_W)
            plsc.store_scatter(idx_v, [pk], val, mask=m)
        carry = carry + jnp.sum(t_chunk)

    bufs = (buf0, buf1)
    sems = (sem0, sem1)
    out_base = w * ROWS_W
    nch = ROWS_W // CH
    copies = [None] * nch

    def start_gather(c):
        copies[c] = pltpu.async_copy(
            x_hbm.at[idx_v.at[pl.ds(c * CH, CH)]], bufs[c % 2], sems[c % 2])

    start_gather(0)
    for c in range(nch):
        if c + 1 < nch:
            start_gather(c + 1)
        copies[c].wait()
        pltpu.sync_copy(bufs[c % 2], out_hbm.at[pl.ds(out_base + c * CH, CH)])


def _dur_body(x_ref, w1_ref, b1_ref, g1_ref, be1_ref,
              w2_ref, b2_ref, g2_ref, be2_ref, lw_ref, lb_ref, out_ref):
    def conv3(h, w_ref, b_ref):
        a0 = jnp.dot(h, w_ref[0], preferred_element_type=jnp.float32,
                     precision=lax.Precision.HIGHEST)
        a1 = jnp.dot(h, w_ref[1], preferred_element_type=jnp.float32,
                     precision=lax.Precision.HIGHEST)
        a2 = jnp.dot(h, w_ref[2], preferred_element_type=jnp.float32,
                     precision=lax.Precision.HIGHEST)
        rows = lax.broadcasted_iota(jnp.int32, (T, F), 0)
        t0 = jnp.where(rows == 0, 0.0, pltpu.roll(a0, 1, 0))
        t2 = jnp.where(rows == T - 1, 0.0, pltpu.roll(a2, -1, 0))
        return a1 + t0 + t2 + b_ref[0]

    def ln(h, g_ref, be_ref):
        mu = jnp.mean(h, axis=1, keepdims=True)
        var = jnp.mean((h - mu) ** 2, axis=1, keepdims=True)
        return (h - mu) / jnp.sqrt(var + 1e-5) * g_ref[0] + be_ref[0]

    h = x_ref[0]
    h = jnp.maximum(ln(conv3(h, w1_ref, b1_ref), g1_ref, be1_ref), 0.0)
    h = jnp.maximum(ln(conv3(h, w2_ref, b2_ref), g2_ref, be2_ref), 0.0)
    dur = lax.dot_general(lw_ref[...], h, (((1,), (1,)), ((), ())),
                          preferred_element_type=jnp.float32,
                          precision=lax.Precision.HIGHEST)
    out_ref[...] = jnp.maximum(dur + lb_ref[...], 0.0)


def _dur_tc(x, c1w, c1b, g1, be1, c2w, c2b, g2, be2, lw, lb):
    vrow = lambda: pl.BlockSpec((1, F), lambda b: (0, 0))
    return pl.pallas_call(
        _dur_body,
        grid=(B,),
        in_specs=[
            pl.BlockSpec((1, T, D), lambda b: (b, 0, 0)),
            pl.BlockSpec((3, D, F), lambda b: (0, 0, 0)),
            vrow(), vrow(), vrow(),
            pl.BlockSpec((3, F, F), lambda b: (0, 0, 0)),
            vrow(), vrow(), vrow(),
            vrow(),
            pl.BlockSpec((1, 1), lambda b: (0, 0)),
        ],
        out_specs=pl.BlockSpec((1, T), lambda b: (b, 0)),
        out_shape=jax.ShapeDtypeStruct((B, T), jnp.float32),
    )(x, c1w, c1b, g1, be1, c2w, c2b, g2, be2, lw, lb)


def kernel(x, target, mel_max_length, conv1_w, conv1_b, ln1_g, ln1_b,
           conv2_w, conv2_b, ln2_g, ln2_b, lin_w, lin_b):
    x_flat = jnp.concatenate(
        [x.reshape(B * T, D), jnp.zeros((8, D), x.dtype)], axis=0)
    out_flat = _expand_sc(x_flat, target)
    output = out_flat.reshape(B, MEL, D)

    dur = _dur_tc(x, conv1_w, conv1_b.reshape(1, F), ln1_g.reshape(1, F),
                  ln1_b.reshape(1, F), conv2_w, conv2_b.reshape(1, F),
                  ln2_g.reshape(1, F), ln2_b.reshape(1, F),
                  lin_w.reshape(1, F), lin_b.reshape(1, 1))
    return (output, dur)


# speedup vs baseline: 37.6794x; 37.6794x over previous
import functools

import jax
import jax.numpy as jnp
from jax import lax
from jax.experimental import pallas as pl
from jax.experimental.pallas import tpu as pltpu
from jax.experimental.pallas import tpu_sc as plsc

B, T, D = 16, 512, 256
F = 256
MEL = 2048

CH = 128
PW = T // 2
FW = MEL // 2
NZ = FW // CH
OUT_PAD = 32


_sc_mesh = plsc.VectorSubcoreMesh(core_axis_name="c", subcore_axis_name="s")


@functools.partial(
    pl.kernel,
    mesh=_sc_mesh,
    out_type=jax.ShapeDtypeStruct((B * MEL + OUT_PAD, D), jnp.float32),
    scratch_types=[
        pltpu.VMEM((T,), jnp.int32),
        pltpu.VMEM((PW, D), jnp.float32),
        pltpu.VMEM((CH, D), jnp.float32),
        pltpu.VMEM((6 + NZ, CH), jnp.int32),
        pltpu.SemaphoreType.DMA,
        pltpu.SemaphoreType.DMA,
    ],
    compiler_params=pltpu.CompilerParams(needs_layout_passes=False),
)
def _expand_sc(x_hbm, tgt_hbm, out_hbm, tgt_v, src_v, zbuf, idx_sc,
               sem0, sem1):
    c = lax.axis_index("c")
    s = lax.axis_index("s")
    w = c * 16 + s
    b = w // 2
    half = w % 2
    p0 = half * PW
    dummy = jnp.int32(B * MEL + w)
    out0 = b * MEL
    iota16 = lax.iota(jnp.int32, 16)

    stage = pltpu.async_copy(x_hbm.at[pl.ds(b * T + p0, PW)], src_v, sem1)

    pltpu.sync_copy(tgt_hbm.at[b], tgt_v)

    def _zrow(j, _):
        for g in range(D // 16):
            zbuf[j, pl.ds(g * 16, 16)] = jnp.zeros((16,), jnp.float32)
        return _
    lax.fori_loop(0, CH, _zrow, 0)

    carry = jnp.int32(0)
    for i in range(T // 16):
        t_chunk = tgt_v[pl.ds(i * 16, 16)]
        csum = plsc.cumsum(t_chunk)
        start = csum - t_chunk + carry
        if i < PW // 16:
            loc = i
            on = half == 0
        else:
            loc = i - PW // 16
            on = half == 1
        pos = out0 + start
        for k in range(3):
            row, lane0 = 2 * k + loc // 8, (loc % 8) * 16
            old = idx_sc[row, pl.ds(lane0, 16)]
            val = jnp.where(t_chunk > k, pos + k, dummy)
            idx_sc[row, pl.ds(lane0, 16)] = jnp.where(on, val, old)
        carry = carry + jnp.sum(t_chunk)

    f0 = half * FW
    for ci in range(NZ):
        for g in range(CH // 16):
            fpos = f0 + ci * CH + g * 16 + iota16
            zval = jnp.where(fpos >= carry, out0 + fpos, dummy)
            idx_sc[6 + ci, pl.ds(g * 16, 16)] = zval

    stage.wait()

    copies = []
    for k in range(3):
        for h in range(2):
            copies.append(pltpu.async_copy(
                src_v.at[pl.ds(h * CH, CH)],
                out_hbm.at[idx_sc.at[2 * k + h]], sem0))
    for ci in range(NZ):
        copies.append(pltpu.async_copy(
            zbuf, out_hbm.at[idx_sc.at[6 + ci]], sem0))
    for cp in copies:
        cp.wait()


def _dur_body(x_ref, w1_ref, b1_ref, g1_ref, be1_ref,
              w2_ref, b2_ref, g2_ref, be2_ref, lw_ref, lb_ref, out_ref):
    def conv3(h, w_ref, b_ref):
        a = jnp.dot(h, w_ref[...], preferred_element_type=jnp.float32,
                    precision=lax.Precision.HIGHEST)
        rows = lax.broadcasted_iota(jnp.int32, (T, F), 0)
        t0 = jnp.where(rows == 0, 0.0, pltpu.roll(a[:, :F], 1, 0))
        t2 = jnp.where(rows == T - 1, 0.0, pltpu.roll(a[:, 2 * F:], T - 1, 0))
        return a[:, F:2 * F] + t0 + t2 + b_ref[0]

    def ln(h, g_ref, be_ref):
        mu = jnp.mean(h, axis=1, keepdims=True)
        var = jnp.mean((h - mu) ** 2, axis=1, keepdims=True)
        return (h - mu) / jnp.sqrt(var + 1e-5) * g_ref[0] + be_ref[0]

    h = x_ref[0]
    h = jnp.maximum(ln(conv3(h, w1_ref, b1_ref), g1_ref, be1_ref), 0.0)
    h = jnp.maximum(ln(conv3(h, w2_ref, b2_ref), g2_ref, be2_ref), 0.0)
    dur = lax.dot_general(lw_ref[...], h, (((1,), (1,)), ((), ())),
                          preferred_element_type=jnp.float32,
                          precision=lax.Precision.HIGHEST)
    out_ref[0] = jnp.maximum(dur + lb_ref[...], 0.0)


def _dur_tc(x, c1w, c1b, g1, be1, c2w, c2b, g2, be2, lw, lb):
    vrow = lambda: pl.BlockSpec((1, F), lambda b: (0, 0))
    return pl.pallas_call(
        _dur_body,
        grid=(B,),
        in_specs=[
            pl.BlockSpec((1, T, D), lambda b: (b, 0, 0)),
            pl.BlockSpec((D, 3 * F), lambda b: (0, 0)),
            vrow(), vrow(), vrow(),
            pl.BlockSpec((F, 3 * F), lambda b: (0, 0)),
            vrow(), vrow(), vrow(),
            vrow(),
            pl.BlockSpec((1, 1), lambda b: (0, 0)),
        ],
        out_specs=pl.BlockSpec((1, 1, T), lambda b: (b, 0, 0)),
        out_shape=jax.ShapeDtypeStruct((B, 1, T), jnp.float32),
    )(x, c1w, c1b, g1, be1, c2w, c2b, g2, be2, lw, lb)


def kernel(x, target, mel_max_length, conv1_w, conv1_b, ln1_g, ln1_b,
           conv2_w, conv2_b, ln2_g, ln2_b, lin_w, lin_b):
    out_flat = _expand_sc(x.reshape(B * T, D), target)
    output = out_flat[:B * MEL].reshape(B, MEL, D)

    w1_cat = conv1_w.transpose(1, 0, 2).reshape(D, 3 * F)
    w2_cat = conv2_w.transpose(1, 0, 2).reshape(F, 3 * F)
    dur = _dur_tc(x, w1_cat, conv1_b.reshape(1, F), ln1_g.reshape(1, F),
                  ln1_b.reshape(1, F), w2_cat, conv2_b.reshape(1, F),
                  ln2_g.reshape(1, F), ln2_b.reshape(1, F),
                  lin_w.reshape(1, F), lin_b.reshape(1, 1))
    return (output, dur.reshape(B, T))


# speedup vs baseline: 45.0617x; 1.1959x over previous
import functools

import jax
import jax.numpy as jnp
from jax import lax
from jax.experimental import pallas as pl
from jax.experimental.pallas import tpu as pltpu
from jax.experimental.pallas import tpu_sc as plsc

B, T, D = 16, 512, 256
F = 256
MEL = 2048

CH = 128
PW = T // 2
FW = MEL // 2
NZ = FW // CH
OUT_PAD = 32


_sc_mesh = plsc.VectorSubcoreMesh(core_axis_name="c", subcore_axis_name="s")


@functools.partial(
    pl.kernel,
    mesh=_sc_mesh,
    out_type=jax.ShapeDtypeStruct((B * MEL + OUT_PAD, D), jnp.float32),
    scratch_types=[
        pltpu.VMEM((T,), jnp.int32),
        pltpu.VMEM((PW, D), jnp.float32),
        pltpu.VMEM((CH, D), jnp.float32),
        pltpu.VMEM((6 + NZ, CH), jnp.int32),
        pltpu.SemaphoreType.DMA,
        pltpu.SemaphoreType.DMA,
    ],
    compiler_params=pltpu.CompilerParams(needs_layout_passes=False),
)
def _expand_sc(x_hbm, tgt_hbm, out_hbm, tgt_v, src_v, zbuf, idx_sc,
               sem0, sem1):
    c = lax.axis_index("c")
    s = lax.axis_index("s")
    w = c * 16 + s
    b = w // 2
    half = w % 2
    p0 = half * PW
    dummy = jnp.int32(B * MEL + w)
    out0 = b * MEL
    iota16 = lax.iota(jnp.int32, 16)

    stage = pltpu.async_copy(x_hbm.at[pl.ds(b * T + p0, PW)], src_v, sem1)

    pltpu.sync_copy(tgt_hbm.at[b], tgt_v)

    def _zrow(j, _):
        for g in range(D // 16):
            zbuf[j, pl.ds(g * 16, 16)] = jnp.zeros((16,), jnp.float32)
        return _
    lax.fori_loop(0, CH, _zrow, 0)

    carry = jnp.int32(0)
    for i in range(T // 16):
        t_chunk = tgt_v[pl.ds(i * 16, 16)]
        csum = plsc.cumsum(t_chunk)
        start = csum - t_chunk + carry
        if i < PW // 16:
            loc = i
            on = half == 0
        else:
            loc = i - PW // 16
            on = half == 1
        pos = out0 + start
        for k in range(3):
            row, lane0 = 2 * k + loc // 8, (loc % 8) * 16
            old = idx_sc[row, pl.ds(lane0, 16)]
            val = jnp.where(t_chunk > k, pos + k, dummy)
            idx_sc[row, pl.ds(lane0, 16)] = jnp.where(on, val, old)
        carry = carry + jnp.sum(t_chunk)

    f0 = half * FW
    for ci in range(NZ):
        for g in range(CH // 16):
            fpos = f0 + ci * CH + g * 16 + iota16
            zval = jnp.where(fpos >= carry, out0 + fpos, dummy)
            idx_sc[6 + ci, pl.ds(g * 16, 16)] = zval

    stage.wait()

    copies = []
    for k in range(3):
        for h in range(2):
            copies.append(pltpu.async_copy(
                src_v.at[pl.ds(h * CH, CH)],
                out_hbm.at[idx_sc.at[2 * k + h]], sem0))

    for ci in range(NZ):
        clo = f0 + ci * CH
        all_pad = carry <= clo
        straddle = jnp.logical_and(clo < carry, carry < clo + CH)

        @pl.when(all_pad)
        def _():
            pltpu.async_copy(zbuf, out_hbm.at[pl.ds(out0 + clo, CH)], sem0)

        @pl.when(straddle)
        def _():
            pltpu.async_copy(zbuf, out_hbm.at[idx_sc.at[6 + ci]], sem0)

    for cp in copies:
        cp.wait()
    for ci in range(NZ):
        clo = f0 + ci * CH
        @pl.when(carry < clo + CH)
        def _():
            pltpu.make_async_copy(
                zbuf, out_hbm.at[pl.ds(out0 + clo, CH)], sem0).wait()


def _dur_body(x_ref, wh1_ref, wl1_ref, b1_ref, g1_ref, be1_ref,
              wh2_ref, wl2_ref, b2_ref, g2_ref, be2_ref, lw_ref, lb_ref,
              out_ref):
    def conv3(h, wh_ref, wl_ref, b_ref):
        hh = h.astype(jnp.bfloat16)
        hl = (h - hh.astype(jnp.float32)).astype(jnp.bfloat16)
        wh = wh_ref[...]
        a = jnp.dot(hh, wh, preferred_element_type=jnp.float32)
        a = a + jnp.dot(hh, wl_ref[...], preferred_element_type=jnp.float32)
        a = a + jnp.dot(hl, wh, preferred_element_type=jnp.float32)
        rows = lax.broadcasted_iota(jnp.int32, (T, F), 0)
        t0 = jnp.where(rows == 0, 0.0, pltpu.roll(a[:, :F], 1, 0))
        t2 = jnp.where(rows == T - 1, 0.0, pltpu.roll(a[:, 2 * F:], T - 1, 0))
        return a[:, F:2 * F] + t0 + t2 + b_ref[0]

    def ln(h, g_ref, be_ref):
        mu = jnp.mean(h, axis=1, keepdims=True)
        var = jnp.mean((h - mu) ** 2, axis=1, keepdims=True)
        return (h - mu) / jnp.sqrt(var + 1e-5) * g_ref[0] + be_ref[0]

    h = x_ref[0]
    h = jnp.maximum(ln(conv3(h, wh1_ref, wl1_ref, b1_ref), g1_ref, be1_ref),
                    0.0)
    h = jnp.maximum(ln(conv3(h, wh2_ref, wl2_ref, b2_ref), g2_ref, be2_ref),
                    0.0)
    dur = lax.dot_general(lw_ref[...], h, (((1,), (1,)), ((), ())),
                          preferred_element_type=jnp.float32,
                          precision=lax.Precision.HIGHEST)
    out_ref[0] = jnp.maximum(dur + lb_ref[...], 0.0)


def _dur_tc(x, wh1, wl1, c1b, g1, be1, wh2, wl2, c2b, g2, be2, lw, lb):
    vrow = lambda: pl.BlockSpec((1, F), lambda b: (0, 0))
    wspec = lambda n: pl.BlockSpec((n, 3 * F), lambda b: (0, 0))
    return pl.pallas_call(
        _dur_body,
        grid=(B,),
        in_specs=[
            pl.BlockSpec((1, T, D), lambda b: (b, 0, 0)),
            wspec(D), wspec(D),
            vrow(), vrow(), vrow(),
            wspec(F), wspec(F),
            vrow(), vrow(), vrow(),
            vrow(),
            pl.BlockSpec((1, 1), lambda b: (0, 0)),
        ],
        out_specs=pl.BlockSpec((1, 1, T), lambda b: (b, 0, 0)),
        out_shape=jax.ShapeDtypeStruct((B, 1, T), jnp.float32),
    )(x, wh1, wl1, c1b, g1, be1, wh2, wl2, c2b, g2, be2, lw, lb)


def kernel(x, target, mel_max_length, conv1_w, conv1_b, ln1_g, ln1_b,
           conv2_w, conv2_b, ln2_g, ln2_b, lin_w, lin_b):
    w1_cat = conv1_w.transpose(1, 0, 2).reshape(D, 3 * F)
    w2_cat = conv2_w.transpose(1, 0, 2).reshape(F, 3 * F)
    wh1 = w1_cat.astype(jnp.bfloat16)
    wl1 = (w1_cat - wh1.astype(jnp.float32)).astype(jnp.bfloat16)
    wh2 = w2_cat.astype(jnp.bfloat16)
    wl2 = (w2_cat - wh2.astype(jnp.float32)).astype(jnp.bfloat16)
    dur = _dur_tc(x, wh1, wl1, conv1_b.reshape(1, F), ln1_g.reshape(1, F),
                  ln1_b.reshape(1, F), wh2, wl2, conv2_b.reshape(1, F),
                  ln2_g.reshape(1, F), ln2_b.reshape(1, F),
                  lin_w.reshape(1, F), lin_b.reshape(1, 1))

    out_flat = _expand_sc(x.reshape(B * T, D), target)
    output = out_flat[:B * MEL].reshape(B, MEL, D)
    return (output, dur.reshape(B, T))
